# baseline (speedup 1.0000x reference)
import functools

import jax
import jax.numpy as jnp
from jax import lax
from jax.experimental import pallas as pl
from jax.experimental.pallas import tpu as pltpu
from jax.experimental.pallas import tpu_sc as plsc

N_NODES = 10000
N_EDGES = 320000
FEAT = 128

NUM_CORES = 2
NUM_SUBCORES = 16
NUM_WORKERS = NUM_CORES * NUM_SUBCORES

CHUNK = 128
CHUNKS_PER_WORKER = 80
HALF = CHUNKS_PER_WORKER // 2
EDGES_PER_WORKER = CHUNKS_PER_WORKER * CHUNK
E_PAD = NUM_WORKERS * EDGES_PER_WORKER

N_PAD = 10240
ROWS_PER_TILE = N_PAD // NUM_SUBCORES


def _sc_agg_body(with_cnt, h_hbm, eidx_hbm, agg_hbm, *rest):
    if with_cnt:
        (cnt_hbm, ebuf, buf0, buf1, ones_v, acc, cnt_acc, sem0, sem1) = rest
    else:
        cnt_hbm = None
        (ebuf, buf0, buf1, ones_v, acc, cnt_acc, sem0, sem1) = rest
    cid = lax.axis_index("c")
    sid = lax.axis_index("s")
    wid = cid * NUM_SUBCORES + sid
    eidx_w = eidx_hbm.at[wid]

    zero16 = jnp.zeros((16,), jnp.float32)
    ones16 = jnp.ones((16,), jnp.float32)

    @pl.loop(0, CHUNK)
    def _(i):
        for j in range(FEAT // 16):
            buf0[i, pl.ds(j * 16, 16)] = zero16

    base = sid * ROWS_PER_TILE
    for k in range(ROWS_PER_TILE // CHUNK):
        pltpu.sync_copy(buf0, acc.at[pl.ds(base + k * CHUNK, CHUNK)])
    if with_cnt:
        for j in range(CHUNK // 16):
            ones_v[pl.ds(j * 16, 16)] = ones16
        for k in range(ROWS_PER_TILE // CHUNK):
            pltpu.sync_copy(
                buf0.at[0],
                cnt_acc.at[pl.ds(base + k * CHUNK, CHUNK)],
            )
    plsc.subcore_barrier()

    def scatter(cl, buf):
        pltpu.sync_copy(buf, acc.at[ebuf.at[cl, 1]], add=True)
        if with_cnt:
            pltpu.sync_copy(ones_v, cnt_acc.at[ebuf.at[cl, 1]], add=True)

    def gather(cl, buf, sem):
        return pltpu.async_copy(h_hbm.at[ebuf.at[cl, 0]], buf, sem)

    def wait_gather(cl, buf, sem):
        pltpu.make_async_copy(h_hbm.at[ebuf.at[cl, 0]], buf, sem).wait()

    pltpu.sync_copy(eidx_w.at[pl.ds(0, HALF)], ebuf)
    for half in range(2):
        gather(0, buf0, sem0)
        gather(1, buf1, sem1)

        @pl.loop(0, HALF // 2 - 1)
        def _(g):
            cl = g * 2
            wait_gather(cl, buf0, sem0)
            scatter(cl, buf0)
            gather(cl + 2, buf0, sem0)
            wait_gather(cl + 1, buf1, sem1)
            scatter(cl + 1, buf1)
            gather(cl + 3, buf1, sem1)

        wait_gather(HALF - 2, buf0, sem0)
        scatter(HALF - 2, buf0)
        wait_gather(HALF - 1, buf1, sem1)
        scatter(HALF - 1, buf1)

        if half == 0:
            pltpu.sync_copy(eidx_w.at[pl.ds(HALF, HALF)], ebuf)

    plsc.subcore_barrier()

    pltpu.sync_copy(acc.at[pl.ds(base, ROWS_PER_TILE)],
                    agg_hbm.at[cid].at[pl.ds(base, ROWS_PER_TILE)])
    if with_cnt:
        pltpu.sync_copy(cnt_acc.at[pl.ds(base, ROWS_PER_TILE)],
                        cnt_hbm.at[cid].at[pl.ds(base, ROWS_PER_TILE)])


def _make_sc_agg(with_cnt):
    outs = [jax.ShapeDtypeStruct((NUM_CORES, N_PAD, FEAT), jnp.float32)]
    if with_cnt:
        outs.append(jax.ShapeDtypeStruct((NUM_CORES, N_PAD), jnp.float32))
    scratch = [
        pltpu.VMEM((HALF, 2, CHUNK), jnp.int32),
        pltpu.VMEM((CHUNK, FEAT), jnp.float32),
        pltpu.VMEM((CHUNK, FEAT), jnp.float32),
        pltpu.VMEM((CHUNK,), jnp.float32),
        pltpu.VMEM_SHARED((N_PAD, FEAT), jnp.float32),
        pltpu.VMEM_SHARED((N_PAD,), jnp.float32),
        pltpu.SemaphoreType.DMA,
        pltpu.SemaphoreType.DMA,
    ]
    body = functools.partial(_sc_agg_body, with_cnt)
    return pl.kernel(
        body,
        out_type=tuple(outs) if with_cnt else outs[0],
        mesh=plsc.VectorSubcoreMesh(core_axis_name="c", subcore_axis_name="s"),
        scratch_types=scratch,
        name="sage_sc_agg" + ("_cnt" if with_cnt else ""),
    )


def _tc_pre_body(h_ref, wr_ref, b_ref, o_ref):
    o_ref[...] = (
        jnp.dot(h_ref[...], wr_ref[...].T, preferred_element_type=jnp.float32)
        + b_ref[...]
    )


_tc_pre = pl.pallas_call(
    _tc_pre_body,
    out_shape=jax.ShapeDtypeStruct((N_NODES, FEAT), jnp.float32),
    name="sage_tc_pre",
)


def _tc_post_body(relu, agg_ref, cnt_ref, pre_ref, wl_ref, o_ref):
    agg = agg_ref[0, :N_NODES, :] + agg_ref[1, :N_NODES, :]
    cnt = cnt_ref[0, :N_NODES] + cnt_ref[1, :N_NODES]
    inv = 1.0 / jnp.maximum(cnt, 1.0)
    mean = agg * inv[:, None]
    out = (
        jnp.dot(mean, wl_ref[...].T, preferred_element_type=jnp.float32)
        + pre_ref[...]
    )
    if relu:
        out = jnp.maximum(out, 0.0)
    o_ref[...] = out


def _make_tc(relu):
    return pl.pallas_call(
        functools.partial(_tc_post_body, relu),
        out_shape=jax.ShapeDtypeStruct((N_NODES, FEAT), jnp.float32),
        name="sage_tc_matmul",
    )


@jax.jit
def kernel(x, edge_index, W1l, b1, W1r, W2l, b2, W2r):
    src = edge_index[0]
    dst = edge_index[1]
    pad = E_PAD - N_EDGES
    pad_ids = jnp.arange(pad, dtype=jnp.int32)
    src_p = jnp.concatenate([src, pad_ids % N_NODES])
    dst_p = jnp.concatenate([dst, N_NODES + pad_ids % (N_PAD - N_NODES)])
    src3 = src_p.reshape(NUM_WORKERS, CHUNKS_PER_WORKER, CHUNK)
    dst3 = dst_p.reshape(NUM_WORKERS, CHUNKS_PER_WORKER, CHUNK)
    eidx = jnp.stack([src3, dst3], axis=2)

    agg1, cnt = _make_sc_agg(True)(x, eidx)
    pre1 = _tc_pre(x, W1r, b1)
    h = _make_tc(True)(agg1, cnt, pre1, W1l)
    agg2 = _make_sc_agg(False)(h, eidx)
    pre2 = _tc_pre(h, W2r, b2)
    return _make_tc(False)(agg2, cnt, pre2, W2l)


# speedup vs baseline: 1.0020x; 1.0020x over previous
import functools

import jax
import jax.numpy as jnp
from jax import lax
from jax.experimental import pallas as pl
from jax.experimental.pallas import tpu as pltpu
from jax.experimental.pallas import tpu_sc as plsc

N_NODES = 10000
N_EDGES = 320000
FEAT = 128

NUM_CORES = 2
NUM_SUBCORES = 16
NUM_WORKERS = NUM_CORES * NUM_SUBCORES

CHUNK = 128
CHUNKS_PER_WORKER = 80
HALF = CHUNKS_PER_WORKER // 2
EDGES_PER_WORKER = CHUNKS_PER_WORKER * CHUNK
E_PAD = NUM_WORKERS * EDGES_PER_WORKER

N_PAD = 10240
ROWS_PER_TILE = N_PAD // NUM_SUBCORES


def _sc_agg_body(with_cnt, h_hbm, eidx_hbm, agg_hbm, *rest):
    if with_cnt:
        (cnt_hbm, ebuf, buf0, buf1, ones_v, acc, cnt_acc, sem0, sem1) = rest
    else:
        cnt_hbm = None
        (ebuf, buf0, buf1, ones_v, acc, cnt_acc, sem0, sem1) = rest
    cid = lax.axis_index("c")
    sid = lax.axis_index("s")
    wid = cid * NUM_SUBCORES + sid
    eidx_w = eidx_hbm.at[wid]

    zero16 = jnp.zeros((16,), jnp.float32)
    ones16 = jnp.ones((16,), jnp.float32)

    @pl.loop(0, CHUNK)
    def _(i):
        for j in range(FEAT // 16):
            buf0[i, pl.ds(j * 16, 16)] = zero16

    base = sid * ROWS_PER_TILE
    for k in range(ROWS_PER_TILE // CHUNK):
        pltpu.sync_copy(buf0, acc.at[pl.ds(base + k * CHUNK, CHUNK)])
    if with_cnt:
        for j in range(CHUNK // 16):
            ones_v[pl.ds(j * 16, 16)] = ones16
        for k in range(ROWS_PER_TILE // CHUNK):
            pltpu.sync_copy(
                buf0.at[0],
                cnt_acc.at[pl.ds(base + k * CHUNK, CHUNK)],
            )
    plsc.subcore_barrier()

    def scatter(cl, buf):
        pltpu.sync_copy(buf, acc.at[ebuf.at[cl, 1]], add=True)
        if with_cnt:
            pltpu.sync_copy(ones_v, cnt_acc.at[ebuf.at[cl, 1]], add=True)

    def gather(cl, buf, sem):
        return pltpu.async_copy(h_hbm.at[ebuf.at[cl, 0]], buf, sem)

    def wait_gather(cl, buf, sem):
        pltpu.make_async_copy(h_hbm.at[ebuf.at[cl, 0]], buf, sem).wait()

    pltpu.sync_copy(eidx_w.at[pl.ds(0, HALF)], ebuf)
    for half in range(2):
        gather(0, buf0, sem0)
        gather(1, buf1, sem1)

        @pl.loop(0, HALF // 2 - 1)
        def _(g):
            cl = g * 2
            wait_gather(cl, buf0, sem0)
            scatter(cl, buf0)
            gather(cl + 2, buf0, sem0)
            wait_gather(cl + 1, buf1, sem1)
            scatter(cl + 1, buf1)
            gather(cl + 3, buf1, sem1)

        wait_gather(HALF - 2, buf0, sem0)
        scatter(HALF - 2, buf0)
        wait_gather(HALF - 1, buf1, sem1)
        scatter(HALF - 1, buf1)

        if half == 0:
            pltpu.sync_copy(eidx_w.at[pl.ds(HALF, HALF)], ebuf)

    plsc.subcore_barrier()

    pltpu.sync_copy(acc.at[pl.ds(base, ROWS_PER_TILE)],
                    agg_hbm.at[cid].at[pl.ds(base, ROWS_PER_TILE)])
    if with_cnt:
        pltpu.sync_copy(cnt_acc.at[pl.ds(base, ROWS_PER_TILE)],
                        cnt_hbm.at[cid].at[pl.ds(base, ROWS_PER_TILE)])


def _make_sc_agg(with_cnt):
    outs = [jax.ShapeDtypeStruct((NUM_CORES, N_PAD, FEAT), jnp.float32)]
    if with_cnt:
        outs.append(jax.ShapeDtypeStruct((NUM_CORES, N_PAD), jnp.float32))
    scratch = [
        pltpu.VMEM((HALF, 2, CHUNK), jnp.int32),
        pltpu.VMEM((CHUNK, FEAT), jnp.float32),
        pltpu.VMEM((CHUNK, FEAT), jnp.float32),
        pltpu.VMEM((CHUNK,), jnp.float32),
        pltpu.VMEM_SHARED((N_PAD, FEAT), jnp.float32),
        pltpu.VMEM_SHARED((N_PAD,), jnp.float32),
        pltpu.SemaphoreType.DMA,
        pltpu.SemaphoreType.DMA,
    ]
    body = functools.partial(_sc_agg_body, with_cnt)
    return pl.kernel(
        body,
        out_type=tuple(outs) if with_cnt else outs[0],
        mesh=plsc.VectorSubcoreMesh(core_axis_name="c", subcore_axis_name="s"),
        scratch_types=scratch,
        name="sage_sc_agg" + ("_cnt" if with_cnt else ""),
    )


def _tc_body(relu, agg_ref, cnt_ref, h_ref, wl_ref, wr_ref, b_ref, o_ref):
    agg = agg_ref[0, :N_NODES, :] + agg_ref[1, :N_NODES, :]
    cnt = cnt_ref[0, :N_NODES] + cnt_ref[1, :N_NODES]
    inv = 1.0 / jnp.maximum(cnt, 1.0)
    mean = agg * inv[:, None]
    out = (
        jnp.dot(mean, wl_ref[...].T, preferred_element_type=jnp.float32)
        + b_ref[...]
        + jnp.dot(h_ref[...], wr_ref[...].T, preferred_element_type=jnp.float32)
    )
    if relu:
        out = jnp.maximum(out, 0.0)
    o_ref[...] = out


def _make_tc(relu):
    return pl.pallas_call(
        functools.partial(_tc_body, relu),
        out_shape=jax.ShapeDtypeStruct((N_NODES, FEAT), jnp.float32),
        name="sage_tc_matmul",
    )


@jax.jit
def kernel(x, edge_index, W1l, b1, W1r, W2l, b2, W2r):
    src = edge_index[0]
    dst = edge_index[1]
    pad = E_PAD - N_EDGES
    pad_ids = jnp.arange(pad, dtype=jnp.int32)
    src_p = jnp.concatenate([src, pad_ids % N_NODES])
    dst_p = jnp.concatenate([dst, N_NODES + pad_ids % (N_PAD - N_NODES)])
    src3 = src_p.reshape(NUM_WORKERS, CHUNKS_PER_WORKER, CHUNK)
    dst3 = dst_p.reshape(NUM_WORKERS, CHUNKS_PER_WORKER, CHUNK)
    eidx = jnp.stack([src3, dst3], axis=2)

    agg1, cnt = _make_sc_agg(True)(x, eidx)
    h = _make_tc(True)(agg1, cnt, x, W1l, W1r, b1)
    agg2 = _make_sc_agg(False)(h, eidx)
    return _make_tc(False)(agg2, cnt, h, W2l, W2r, b2)
